# Initial kernel scaffold; baseline (speedup 1.0000x reference)
import jax
import jax.numpy as jnp
from jax.experimental import pallas as pl


def kernel(embeds, centroids, r):
    raise NotImplementedError("write your pallas kernel here")


# speedup vs baseline: 264.9466x; 264.9466x over previous
import jax
import jax.numpy as jnp
from jax.experimental import pallas as pl
from jax.experimental.pallas import tpu as pltpu

_B, _N, _D, _P = 4, 3136, 64, 8192
_H = 56
_QT = 256
_PT = 2048


def _min_dist_kernel(q_ref, ct_ref, out_ref):
    j = pl.program_id(1)
    q = q_ref[...]
    ct = ct_ref[...]
    dots = jnp.dot(q, ct, preferred_element_type=jnp.float32)
    cnorm = jnp.sum(ct * ct, axis=0)
    m = jnp.min(cnorm[None, :] - 2.0 * dots, axis=1, keepdims=True)

    @pl.when(j == 0)
    def _():
        out_ref[...] = m

    @pl.when(j > 0)
    def _():
        out_ref[...] = jnp.minimum(out_ref[...], m)

    @pl.when(j == _P // _PT - 1)
    def _():
        qnorm = jnp.sum(q * q, axis=1, keepdims=True)
        out_ref[...] = jnp.sqrt(out_ref[...] + qnorm)


@jax.jit
def kernel(embeds, centroids, r):
    del r
    q = embeds.reshape(_B * _N, _D)
    ct = centroids.T
    out = pl.pallas_call(
        _min_dist_kernel,
        grid=(_B * _N // _QT, _P // _PT),
        in_specs=[
            pl.BlockSpec((_QT, _D), lambda i, j: (i, 0)),
            pl.BlockSpec((_D, _PT), lambda i, j: (0, j)),
        ],
        out_specs=pl.BlockSpec((_QT, 1), lambda i, j: (i, 0)),
        out_shape=jax.ShapeDtypeStruct((_B * _N, 1), jnp.float32),
        compiler_params=pltpu.CompilerParams(
            dimension_semantics=("parallel", "arbitrary")),
    )(q, ct)
    score = jnp.transpose(out.reshape(_B, _H, _H, 1), (0, 3, 1, 2))
    return (jnp.float32(0.0), score)


# speedup vs baseline: 584.6069x; 2.2065x over previous
import jax
import jax.numpy as jnp
from jax.experimental import pallas as pl
from jax.experimental.pallas import tpu as pltpu

_B, _N, _D, _P = 4, 3136, 64, 8192
_H = 56
_QT = 256
_PT = 2048


def _min_dist_kernel(q_ref, ct_ref, out_ref):
    q = q_ref[...]
    ct = ct_ref[...]
    dots = jnp.dot(q, ct, preferred_element_type=jnp.float32)
    cnorm = jnp.sum(ct * ct, axis=0)
    m = jnp.min(cnorm[None, :] - 2.0 * dots, axis=1, keepdims=True)
    qnorm = jnp.sum(q * q, axis=1, keepdims=True)
    out_ref[...] = jnp.sqrt(m + qnorm)


@jax.jit
def kernel(embeds, centroids, r):
    del r
    q = embeds.reshape(_B * _N, _D)
    ct = centroids.T
    out = pl.pallas_call(
        _min_dist_kernel,
        grid=(_B * _N // _QT,),
        in_specs=[
            pl.BlockSpec((_QT, _D), lambda i: (i, 0)),
            pl.BlockSpec((_D, _P), lambda i: (0, 0)),
        ],
        out_specs=pl.BlockSpec((_QT, 1), lambda i: (i, 0)),
        out_shape=jax.ShapeDtypeStruct((_B * _N, 1), jnp.float32),
        compiler_params=pltpu.CompilerParams(
            dimension_semantics=("arbitrary",)),
    )(q, ct)
    score = jnp.transpose(out.reshape(_B, _H, _H, 1), (0, 3, 1, 2))
    return (jnp.float32(0.0), score)


# speedup vs baseline: 628.6000x; 1.0753x over previous
import jax
import jax.numpy as jnp
from jax.experimental import pallas as pl
from jax.experimental.pallas import tpu as pltpu

_B, _N, _D, _P = 4, 3136, 64, 8192
_H = 56
_QT = 448
_PT = 2048


def _min_dist_kernel(q_ref, ct_ref, out_ref):
    q = q_ref[...]
    ct = ct_ref[...]
    dots = jnp.dot(q, ct, preferred_element_type=jnp.float32)
    cnorm = jnp.sum(ct * ct, axis=0)
    m = jnp.min(cnorm[None, :] - 2.0 * dots, axis=1, keepdims=True)
    qnorm = jnp.sum(q * q, axis=1, keepdims=True)
    out_ref[...] = jnp.sqrt(m + qnorm)


@jax.jit
def kernel(embeds, centroids, r):
    del r
    q = embeds.reshape(_B * _N, _D)
    ct = centroids.T
    out = pl.pallas_call(
        _min_dist_kernel,
        grid=(_B * _N // _QT,),
        in_specs=[
            pl.BlockSpec((_QT, _D), lambda i: (i, 0)),
            pl.BlockSpec((_D, _P), lambda i: (0, 0)),
        ],
        out_specs=pl.BlockSpec((_QT, 1), lambda i: (i, 0)),
        out_shape=jax.ShapeDtypeStruct((_B * _N, 1), jnp.float32),
        compiler_params=pltpu.CompilerParams(
            dimension_semantics=("parallel",)),
    )(q, ct)
    score = jnp.transpose(out.reshape(_B, _H, _H, 1), (0, 3, 1, 2))
    return (jnp.float32(0.0), score)
